# baseline (speedup 1.0000x reference)
import functools

import jax
import jax.numpy as jnp
from jax import lax
from jax.experimental import pallas as pl
from jax.experimental.pallas import tpu as pltpu
from jax.experimental.pallas import tpu_sc as plsc

N = 10000
E = 320000
D = 128
EPS = 1e-5

NC = 2
NS = 16
NW = NC * NS
L = 16

EPW = E // NW
C = 80
NCHUNK = EPW // C
ZCH = 80
NRCH = N // ZCH
KMAX = -(-NRCH // NS)

_SC_PARAMS = pltpu.CompilerParams(needs_layout_passes=False)


def _dense_body(act, x_ref, wcat_ref, fb_ref, self_ref, h_ref, bg_ref, p_ref):
    x = x_ref[...]
    y = jnp.dot(x, wcat_ref[...], preferred_element_type=jnp.float32)
    swy = y[:, :D]
    h = y[:, D:2 * D]
    bg = y[:, 2 * D:4 * D] + fb_ref[...]
    bgs = y[:, 4 * D:]
    beta_s, gamma_s = bgs[:, :D], bgs[:, D:]
    out = gamma_s * swy + beta_s
    if act:
        out = jnp.maximum(out, 0.0)
    self_ref[...] = out
    h_ref[...] = h
    bg_ref[...] = bg
    beta, gamma = bg[:, :D], bg[:, D:]
    b16 = lax.bitcast_convert_type(beta.astype(jnp.bfloat16),
                                   jnp.uint16).astype(jnp.uint32)
    g16 = lax.bitcast_convert_type(gamma.astype(jnp.bfloat16),
                                   jnp.uint16).astype(jnp.uint32)
    p_ref[...] = lax.bitcast_convert_type(b16 | (g16 << 16), jnp.int32)


def _dense(x, wcat, fb, act):
    blk = 1000
    grid = N // blk
    return pl.pallas_call(
        functools.partial(_dense_body, act),
        grid=(grid,),
        in_specs=[
            pl.BlockSpec((blk, D), lambda i: (i, 0)),
            pl.BlockSpec((D, 6 * D), lambda i: (0, 0)),
            pl.BlockSpec((1, 2 * D), lambda i: (0, 0)),
        ],
        out_specs=[
            pl.BlockSpec((blk, D), lambda i: (i, 0)),
            pl.BlockSpec((blk, D), lambda i: (i, 0)),
            pl.BlockSpec((blk, 2 * D), lambda i: (i, 0)),
            pl.BlockSpec((blk, D), lambda i: (i, 0)),
        ],
        out_shape=[
            jax.ShapeDtypeStruct((N, D), jnp.float32),
            jax.ShapeDtypeStruct((N, D), jnp.float32),
            jax.ShapeDtypeStruct((N, 2 * D), jnp.float32),
            jax.ShapeDtypeStruct((N, D), jnp.int32),
        ],
    )(x, wcat, fb)


def _combine_bn0_body(self_ref, agg_ref, cntp_ref, g_ref, b_ref,
                      out_ref, cnt_ref):
    cnt = jnp.sum(cntp_ref[...], axis=0)
    recip = 1.0 / jnp.clip(cnt, 1.0, None)
    agg = (agg_ref[0, :, :] + agg_ref[1, :, :]) * recip[:, None]
    t = self_ref[...] + agg
    m = jnp.mean(t, axis=0)
    v = jnp.mean((t - m) ** 2, axis=0)
    out_ref[...] = g_ref[...] * (t - m) / jnp.sqrt(v + EPS) + b_ref[...]
    cnt_ref[...] = jnp.broadcast_to(cnt[:, None], (N, D))


def _combine_bn0(selfo, agg, cnt_parts, g, b):
    return pl.pallas_call(
        _combine_bn0_body,
        out_shape=[
            jax.ShapeDtypeStruct((N, D), jnp.float32),
            jax.ShapeDtypeStruct((N, D), jnp.float32),
        ],
    )(selfo, agg, cnt_parts, g, b)


def _combine_bn1_body(self_ref, agg_ref, cnt_ref, g_ref, b_ref, out_ref):
    recip = 1.0 / jnp.clip(cnt_ref[...], 1.0, None)
    t = self_ref[...] + (agg_ref[0, :, :] + agg_ref[1, :, :]) * recip
    m = jnp.mean(t, axis=0)
    v = jnp.mean((t - m) ** 2, axis=0)
    out_ref[...] = g_ref[...] * (t - m) / jnp.sqrt(v + EPS) + b_ref[...]


def _combine_bn1(selfo, agg, cnt_bc, g, b):
    return pl.pallas_call(
        _combine_bn1_body,
        out_shape=jax.ShapeDtypeStruct((N, D), jnp.float32),
    )(selfo, agg, cnt_bc, g, b)


def _combine2_body(self_ref, agg_ref, cnt_ref, bg_ref, out_ref):
    cnt = cnt_ref[...]
    recip = 1.0 / jnp.clip(cnt, 1.0, None)
    ind = (cnt > 0.0).astype(jnp.float32)
    s = (agg_ref[0, :, :] + agg_ref[1, :, :]) * recip
    beta, gamma = bg_ref[:, :D], bg_ref[:, D:]
    out_ref[...] = self_ref[...] + gamma * s + beta * ind


def _combine2(selfo, agg, cnt_bc, bg):
    return pl.pallas_call(
        _combine2_body,
        out_shape=jax.ShapeDtypeStruct((N, D), jnp.float32),
    )(selfo, agg, cnt_bc, bg)


_MESH = plsc.VectorSubcoreMesh(core_axis_name="c", subcore_axis_name="s")


def _zero_vmem_2d(ref, rows):
    def body(i, _):
        for j in range(D // L):
            ref[i, pl.ds(j * L, L)] = jnp.zeros((L,), jnp.float32)
        return 0
    lax.fori_loop(0, rows, body, 0)


def _zero_spmem(zsrc, aggsh, sid):
    for k in range(KMAX):
        rc = k * NS + sid

        @pl.when(rc < NRCH)
        def _():
            pltpu.sync_copy(zsrc, aggsh.at[pl.ds(rc * ZCH, ZCH)])


def _write_out(aggsh, agg_hbm, cid, sid):
    for k in range(KMAX):
        rc = k * NS + sid

        @pl.when(rc < NRCH)
        def _():
            r0 = rc * ZCH
            pltpu.sync_copy(aggsh.at[pl.ds(r0, ZCH)],
                            agg_hbm.at[cid, pl.ds(r0, ZCH)])


def _edge_film_body(h_hbm, p_hbm, src_hbm, dst_hbm, agg_hbm,
                    sidxr, didxr, h0, h1, p0, p1, aggsh,
                    semh0, semh1, semp0, semp1, semi0, semi1, sems0, sems1):
    cid = lax.axis_index("c")
    sid = lax.axis_index("s")
    wid = sid * NC + cid
    H, P = (h0, h1), (p0, p1)
    SEMH, SEMP = (semh0, semh1), (semp0, semp1)
    SEMI, SEMS = (semi0, semi1), (sems0, sems1)

    def base_of(k):
        return pl.multiple_of(wid * EPW + k * C, 8)

    def issue_idx(k, b):
        r = k % 4
        pltpu.async_copy(src_hbm.at[pl.ds(base_of(k), C)], sidxr.at[r],
                         SEMI[b])
        pltpu.async_copy(dst_hbm.at[pl.ds(base_of(k), C)], didxr.at[r],
                         SEMI[b])

    def wait_idx(k, b):
        r = k % 4
        pltpu.make_async_copy(src_hbm.at[pl.ds(base_of(k), C)], sidxr.at[r],
                              SEMI[b]).wait()
        pltpu.make_async_copy(dst_hbm.at[pl.ds(base_of(k), C)], didxr.at[r],
                              SEMI[b]).wait()

    def issue_gather(k, b):
        r = k % 4
        pltpu.async_copy(h_hbm.at[sidxr.at[r]], H[b], SEMH[b])
        pltpu.async_copy(p_hbm.at[didxr.at[r]], P[b], SEMP[b])

    def wait_gather(k, b):
        r = k % 4
        pltpu.make_async_copy(h_hbm.at[sidxr.at[r]], H[b], SEMH[b]).wait()
        pltpu.make_async_copy(p_hbm.at[didxr.at[r]], P[b], SEMP[b]).wait()

    def issue_scatter(k, b):
        pltpu.async_copy(H[b], aggsh.at[didxr.at[k % 4]], SEMS[b], add=True)

    def wait_scatter(k, b):
        pltpu.make_async_copy(H[b], aggsh.at[didxr.at[k % 4]],
                              SEMS[b]).wait()

    def compute(b):
        @plsc.parallel_loop(0, C, step=1, unroll=4)
        def _(i):
            for j in range(D // L):
                pw = P[b][i, pl.ds(L * j, L)]
                pj = plsc.bitcast(pw, jnp.bfloat16)
                bb, gg = plsc.unpack(pj, format=plsc.PackFormat.INTERLEAVED,
                                     preferred_element_type=jnp.float32)
                hh = H[b][i, pl.ds(L * j, L)]
                H[b][i, pl.ds(L * j, L)] = jnp.maximum(gg * hh + bb, 0.0)

    def step(k, b, first):
        wait_gather(k, b)
        compute(b)
        issue_scatter(k, b)

        @pl.when(k + 2 < NCHUNK)
        def _():
            issue_idx(k + 2, b)
        if not first:
            wait_scatter(k - 1, 1 - b)
        wait_idx(k + 1, 1 - b)
        issue_gather(k + 1, 1 - b)

    _zero_vmem_2d(h0, C)
    _zero_spmem(h0, aggsh, sid)
    plsc.subcore_barrier()

    issue_idx(0, 0)
    issue_idx(1, 1)
    wait_idx(0, 0)
    issue_gather(0, 0)
    step(0, 0, True)
    step(1, 1, False)

    def pair(kk, _):
        k = 2 * kk
        step(k, 0, False)
        step(k + 1, 1, False)
        return 0

    lax.fori_loop(1, NCHUNK // 2, pair, 0)
    wait_gather(NCHUNK - 1, 0)
    compute(0)
    wait_scatter(NCHUNK - 2, 1)
    pltpu.sync_copy(H[0], aggsh.at[didxr.at[(NCHUNK - 1) % 4]], add=True)

    plsc.subcore_barrier()
    _write_out(aggsh, agg_hbm, cid, sid)


def _edge_film(h, p, src, dst):
    return pl.kernel(
        _edge_film_body,
        out_type=jax.ShapeDtypeStruct((NC, N, D), jnp.float32),
        mesh=_MESH,
        compiler_params=_SC_PARAMS,
        scratch_types=[
            pltpu.VMEM((4, C), jnp.int32),
            pltpu.VMEM((4, C), jnp.int32),
            pltpu.VMEM((C, D), jnp.float32),
            pltpu.VMEM((C, D), jnp.float32),
            pltpu.VMEM((C, D), jnp.int32),
            pltpu.VMEM((C, D), jnp.int32),
            pltpu.VMEM_SHARED((N, D), jnp.float32),
            pltpu.SemaphoreType.DMA,
            pltpu.SemaphoreType.DMA,
            pltpu.SemaphoreType.DMA,
            pltpu.SemaphoreType.DMA,
            pltpu.SemaphoreType.DMA,
            pltpu.SemaphoreType.DMA,
            pltpu.SemaphoreType.DMA,
            pltpu.SemaphoreType.DMA,
        ],
    )(h, p, src, dst)


def _edge_sum_body(h_hbm, src_hbm, dst_hbm, agg_hbm,
                   sidxr, didxr, h0, h1, aggsh,
                   semh0, semh1, semi0, semi1, sems0, sems1):
    cid = lax.axis_index("c")
    sid = lax.axis_index("s")
    wid = sid * NC + cid
    H = (h0, h1)
    SEMH, SEMI, SEMS = (semh0, semh1), (semi0, semi1), (sems0, sems1)

    def base_of(k):
        return pl.multiple_of(wid * EPW + k * C, 8)

    def issue_idx(k, b):
        r = k % 4
        pltpu.async_copy(src_hbm.at[pl.ds(base_of(k), C)], sidxr.at[r],
                         SEMI[b])
        pltpu.async_copy(dst_hbm.at[pl.ds(base_of(k), C)], didxr.at[r],
                         SEMI[b])

    def wait_idx(k, b):
        r = k % 4
        pltpu.make_async_copy(src_hbm.at[pl.ds(base_of(k), C)], sidxr.at[r],
                              SEMI[b]).wait()
        pltpu.make_async_copy(dst_hbm.at[pl.ds(base_of(k), C)], didxr.at[r],
                              SEMI[b]).wait()

    def issue_gather(k, b):
        pltpu.async_copy(h_hbm.at[sidxr.at[k % 4]], H[b], SEMH[b])

    def wait_gather(k, b):
        pltpu.make_async_copy(h_hbm.at[sidxr.at[k % 4]], H[b],
                              SEMH[b]).wait()

    def issue_scatter(k, b):
        pltpu.async_copy(H[b], aggsh.at[didxr.at[k % 4]], SEMS[b], add=True)

    def wait_scatter(k, b):
        pltpu.make_async_copy(H[b], aggsh.at[didxr.at[k % 4]],
                              SEMS[b]).wait()

    def step(k, b, first):
        wait_gather(k, b)
        issue_scatter(k, b)

        @pl.when(k + 2 < NCHUNK)
        def _():
            issue_idx(k + 2, b)
        if not first:
            wait_scatter(k - 1, 1 - b)
        wait_idx(k + 1, 1 - b)
        issue_gather(k + 1, 1 - b)

    _zero_vmem_2d(h0, C)
    _zero_spmem(h0, aggsh, sid)
    plsc.subcore_barrier()

    issue_idx(0, 0)
    issue_idx(1, 1)
    wait_idx(0, 0)
    issue_gather(0, 0)
    step(0, 0, True)
    step(1, 1, False)

    def pair(kk, _):
        k = 2 * kk
        step(k, 0, False)
        step(k + 1, 1, False)
        return 0

    lax.fori_loop(1, NCHUNK // 2, pair, 0)
    wait_gather(NCHUNK - 1, 0)
    wait_scatter(NCHUNK - 2, 1)
    pltpu.sync_copy(H[0], aggsh.at[didxr.at[(NCHUNK - 1) % 4]], add=True)

    plsc.subcore_barrier()
    _write_out(aggsh, agg_hbm, cid, sid)


def _edge_sum(h, src, dst):
    return pl.kernel(
        _edge_sum_body,
        out_type=jax.ShapeDtypeStruct((NC, N, D), jnp.float32),
        mesh=_MESH,
        compiler_params=_SC_PARAMS,
        scratch_types=[
            pltpu.VMEM((4, C), jnp.int32),
            pltpu.VMEM((4, C), jnp.int32),
            pltpu.VMEM((C, D), jnp.float32),
            pltpu.VMEM((C, D), jnp.float32),
            pltpu.VMEM_SHARED((N, D), jnp.float32),
            pltpu.SemaphoreType.DMA,
            pltpu.SemaphoreType.DMA,
            pltpu.SemaphoreType.DMA,
            pltpu.SemaphoreType.DMA,
            pltpu.SemaphoreType.DMA,
            pltpu.SemaphoreType.DMA,
        ],
    )(h, src, dst)


def _cnt_body(dst_hbm, cnt_hbm, didx, cntv):
    cid = lax.axis_index("c")
    sid = lax.axis_index("s")
    wid = sid * NC + cid

    pltpu.sync_copy(dst_hbm.at[pl.ds(pl.multiple_of(wid * EPW, 8), EPW)], didx)

    def zc(i, _):
        cntv[pl.ds(i * L, L)] = jnp.zeros((L,), jnp.float32)
        return 0
    lax.fori_loop(0, N // L, zc, 0)

    ones16 = jnp.ones((L,), jnp.float32)

    def acc(g, _):
        plsc.addupdate_scatter(cntv, [didx[pl.ds(g * L, L)]], ones16)
        return 0
    lax.fori_loop(0, EPW // L, acc, 0)

    pltpu.sync_copy(cntv, cnt_hbm.at[wid, 0])


def _cnt(dst):
    return pl.kernel(
        _cnt_body,
        out_type=jax.ShapeDtypeStruct((NW, 1, N), jnp.float32),
        mesh=_MESH,
        compiler_params=_SC_PARAMS,
        scratch_types=[
            pltpu.VMEM((EPW,), jnp.int32),
            pltpu.VMEM((N,), jnp.float32),
        ],
    )(dst)


def kernel(x, edge_index, W0, Fw0, Fb0, Sw0, FSw0, W1, Fw1, Fb1, Sw1, FSw1,
           W2, Fw2, Fb2, Sw2, FSw2, bng0, bnb0, bng1, bnb1):
    src, dst = edge_index[0], edge_index[1]

    wcat0 = jnp.concatenate([Sw0, W0, Fw0, FSw0], axis=1)
    wcat1 = jnp.concatenate([Sw1, W1, Fw1, FSw1], axis=1)
    wcat2 = jnp.concatenate([Sw2, W2, Fw2, FSw2], axis=1)

    cnt_parts = _cnt(dst).reshape(NW, N)

    self0, h0, bg0, p0 = _dense(x, wcat0, Fb0.reshape(1, -1), act=True)
    agg0 = _edge_film(h0, p0, src, dst)
    x1, cnt_bc = _combine_bn0(self0, agg0, cnt_parts,
                              bng0.reshape(1, -1), bnb0.reshape(1, -1))

    self1, h1, bg1, p1 = _dense(x1, wcat1, Fb1.reshape(1, -1), act=True)
    agg1 = _edge_film(h1, p1, src, dst)
    x2 = _combine_bn1(self1, agg1, cnt_bc,
                      bng1.reshape(1, -1), bnb1.reshape(1, -1))

    self2, h2, bg2, p2 = _dense(x2, wcat2, Fb2.reshape(1, -1), act=False)
    agg2 = _edge_sum(h2, src, dst)
    return _combine2(self2, agg2, cnt_bc, bg2)


# speedup vs baseline: 1.0140x; 1.0140x over previous
import functools

import jax
import jax.numpy as jnp
from jax import lax
from jax.experimental import pallas as pl
from jax.experimental.pallas import tpu as pltpu
from jax.experimental.pallas import tpu_sc as plsc

N = 10000
E = 320000
D = 128
EPS = 1e-5

NC = 2
NS = 16
NW = NC * NS
L = 16

EPW = E // NW
C = 80
NCHUNK = EPW // C
ZCH = 80
NRCH = N // ZCH
KMAX = -(-NRCH // NS)

_SC_PARAMS = pltpu.CompilerParams(needs_layout_passes=False)


def _dense_body(act, x_ref, wcat_ref, fb_ref, self_ref, h_ref, bg_ref, p_ref):
    x = x_ref[...]
    y = jnp.dot(x, wcat_ref[...], preferred_element_type=jnp.float32)
    swy = y[:, :D]
    h = y[:, D:2 * D]
    bg = y[:, 2 * D:4 * D] + fb_ref[...]
    bgs = y[:, 4 * D:]
    beta_s, gamma_s = bgs[:, :D], bgs[:, D:]
    out = gamma_s * swy + beta_s
    if act:
        out = jnp.maximum(out, 0.0)
    self_ref[...] = out
    h_ref[...] = h
    bg_ref[...] = bg
    beta, gamma = bg[:, :D], bg[:, D:]
    b16 = lax.bitcast_convert_type(beta.astype(jnp.bfloat16),
                                   jnp.uint16).astype(jnp.uint32)
    g16 = lax.bitcast_convert_type(gamma.astype(jnp.bfloat16),
                                   jnp.uint16).astype(jnp.uint32)
    p_ref[...] = lax.bitcast_convert_type(b16 | (g16 << 16), jnp.float32)


def _dense(x, wcat, fb, act):
    blk = 1000
    grid = N // blk
    return pl.pallas_call(
        functools.partial(_dense_body, act),
        grid=(grid,),
        in_specs=[
            pl.BlockSpec((blk, D), lambda i: (i, 0)),
            pl.BlockSpec((D, 6 * D), lambda i: (0, 0)),
            pl.BlockSpec((1, 2 * D), lambda i: (0, 0)),
        ],
        out_specs=[
            pl.BlockSpec((blk, D), lambda i: (i, 0)),
            pl.BlockSpec((blk, D), lambda i: (i, 0)),
            pl.BlockSpec((blk, 2 * D), lambda i: (i, 0)),
            pl.BlockSpec((blk, D), lambda i: (i, 0)),
        ],
        out_shape=[
            jax.ShapeDtypeStruct((N, D), jnp.float32),
            jax.ShapeDtypeStruct((N, D), jnp.float32),
            jax.ShapeDtypeStruct((N, 2 * D), jnp.float32),
            jax.ShapeDtypeStruct((N, D), jnp.float32),
        ],
    )(x, wcat, fb)


def _fused_body(act, self_ref, agg_ref, cntp_ref, g_ref, b_ref,
                wcat_ref, fb_ref, self_o, h_o, bg_o, p_o):
    cnt = jnp.sum(cntp_ref[...], axis=0)
    recip = 1.0 / jnp.clip(cnt, 1.0, None)
    t = self_ref[...] + (agg_ref[0, :, :] + agg_ref[1, :, :]) * recip[:, None]
    m = jnp.mean(t, axis=0)
    v = jnp.mean((t - m) ** 2, axis=0)
    xn = g_ref[...] * (t - m) / jnp.sqrt(v + EPS) + b_ref[...]
    blk = 1000
    for i in range(N // blk):
        rows = pl.ds(i * blk, blk)
        y = jnp.dot(xn[i * blk:(i + 1) * blk, :], wcat_ref[...],
                    preferred_element_type=jnp.float32)
        swy = y[:, :D]
        h = y[:, D:2 * D]
        bg = y[:, 2 * D:4 * D] + fb_ref[...]
        bgs = y[:, 4 * D:]
        beta_s, gamma_s = bgs[:, :D], bgs[:, D:]
        out = gamma_s * swy + beta_s
        if act:
            out = jnp.maximum(out, 0.0)
        self_o[rows, :] = out
        h_o[rows, :] = h
        bg_o[rows, :] = bg
        beta, gamma = bg[:, :D], bg[:, D:]
        b16 = lax.bitcast_convert_type(beta.astype(jnp.bfloat16),
                                       jnp.uint16).astype(jnp.uint32)
        g16 = lax.bitcast_convert_type(gamma.astype(jnp.bfloat16),
                                       jnp.uint16).astype(jnp.uint32)
        p_o[rows, :] = lax.bitcast_convert_type(b16 | (g16 << 16),
                                                jnp.float32)


def _fused(selfo, agg, cnt_parts, g, b, wcat, fb, act):
    return pl.pallas_call(
        functools.partial(_fused_body, act),
        out_shape=[
            jax.ShapeDtypeStruct((N, D), jnp.float32),
            jax.ShapeDtypeStruct((N, D), jnp.float32),
            jax.ShapeDtypeStruct((N, 2 * D), jnp.float32),
            jax.ShapeDtypeStruct((N, D), jnp.float32),
        ],
    )(selfo, agg, cnt_parts, g, b, wcat, fb)


def _combine2_body(self_ref, agg_ref, cntp_ref, bg_ref, out_ref):
    cnt = jnp.sum(cntp_ref[...], axis=0)
    recip = 1.0 / jnp.clip(cnt, 1.0, None)
    ind = (cnt > 0.0).astype(jnp.float32)
    s = (agg_ref[0, :, :] + agg_ref[1, :, :]) * recip[:, None]
    beta, gamma = bg_ref[:, :D], bg_ref[:, D:]
    out_ref[...] = self_ref[...] + gamma * s + beta * ind[:, None]


def _combine2(selfo, agg, cnt_parts, bg):
    return pl.pallas_call(
        _combine2_body,
        out_shape=jax.ShapeDtypeStruct((N, D), jnp.float32),
    )(selfo, agg, cnt_parts, bg)


_MESH = plsc.VectorSubcoreMesh(core_axis_name="c", subcore_axis_name="s")


def _zero_vmem_2d(ref, rows):
    def body(i, _):
        for j in range(D // L):
            ref[i, pl.ds(j * L, L)] = jnp.zeros((L,), jnp.float32)
        return 0
    lax.fori_loop(0, rows, body, 0)


def _zero_spmem(zsrc, aggsh, sid):
    for k in range(KMAX):
        rc = k * NS + sid

        @pl.when(rc < NRCH)
        def _():
            pltpu.sync_copy(zsrc, aggsh.at[pl.ds(rc * ZCH, ZCH)])


def _write_out(aggsh, agg_hbm, cid, sid):
    for k in range(KMAX):
        rc = k * NS + sid

        @pl.when(rc < NRCH)
        def _():
            r0 = rc * ZCH
            pltpu.sync_copy(aggsh.at[pl.ds(r0, ZCH)],
                            agg_hbm.at[cid, pl.ds(r0, ZCH)])


def _edge_film_body(h_hbm, p_hbm, src_hbm, dst_hbm, agg_hbm,
                    sidxr, didxr, h0, h1, p0, p1, aggsh,
                    semh0, semh1, semp0, semp1, semi0, semi1, sems0, sems1):
    cid = lax.axis_index("c")
    sid = lax.axis_index("s")
    wid = sid * NC + cid
    H, P = (h0, h1), (p0, p1)
    SEMH, SEMP = (semh0, semh1), (semp0, semp1)
    SEMI, SEMS = (semi0, semi1), (sems0, sems1)

    def base_of(k):
        return pl.multiple_of(wid * EPW + k * C, 8)

    def issue_idx(k, b):
        r = k % 4
        pltpu.async_copy(src_hbm.at[pl.ds(base_of(k), C)], sidxr.at[r],
                         SEMI[b])
        pltpu.async_copy(dst_hbm.at[pl.ds(base_of(k), C)], didxr.at[r],
                         SEMI[b])

    def wait_idx(k, b):
        r = k % 4
        pltpu.make_async_copy(src_hbm.at[pl.ds(base_of(k), C)], sidxr.at[r],
                              SEMI[b]).wait()
        pltpu.make_async_copy(dst_hbm.at[pl.ds(base_of(k), C)], didxr.at[r],
                              SEMI[b]).wait()

    def issue_gather(k, b):
        r = k % 4
        pltpu.async_copy(h_hbm.at[sidxr.at[r]], H[b], SEMH[b])
        pltpu.async_copy(p_hbm.at[didxr.at[r]], P[b], SEMP[b])

    def wait_gather(k, b):
        r = k % 4
        pltpu.make_async_copy(h_hbm.at[sidxr.at[r]], H[b], SEMH[b]).wait()
        pltpu.make_async_copy(p_hbm.at[didxr.at[r]], P[b], SEMP[b]).wait()

    def issue_scatter(k, b):
        pltpu.async_copy(H[b], aggsh.at[didxr.at[k % 4]], SEMS[b], add=True)

    def wait_scatter(k, b):
        pltpu.make_async_copy(H[b], aggsh.at[didxr.at[k % 4]],
                              SEMS[b]).wait()

    def compute(b):
        @plsc.parallel_loop(0, C, step=1, unroll=2)
        def _(i):
            for j in range(D // L):
                pw = P[b][i, pl.ds(L * j, L)]
                pj = plsc.bitcast(pw, jnp.bfloat16)
                bb, gg = plsc.unpack(pj, format=plsc.PackFormat.INTERLEAVED,
                                     preferred_element_type=jnp.float32)
                hh = H[b][i, pl.ds(L * j, L)]
                H[b][i, pl.ds(L * j, L)] = jnp.maximum(gg * hh + bb, 0.0)

    def step(k, b, first):
        wait_gather(k, b)
        compute(b)
        issue_scatter(k, b)

        @pl.when(k + 2 < NCHUNK)
        def _():
            issue_idx(k + 2, b)
        if not first:
            wait_scatter(k - 1, 1 - b)
        wait_idx(k + 1, 1 - b)
        issue_gather(k + 1, 1 - b)

    _zero_vmem_2d(h0, C)
    _zero_spmem(h0, aggsh, sid)
    plsc.subcore_barrier()

    issue_idx(0, 0)
    issue_idx(1, 1)
    wait_idx(0, 0)
    issue_gather(0, 0)
    step(0, 0, True)
    step(1, 1, False)

    def pair(kk, _):
        k = 2 * kk
        step(k, 0, False)
        step(k + 1, 1, False)
        return 0

    lax.fori_loop(1, NCHUNK // 2, pair, 0)
    wait_gather(NCHUNK - 1, 0)
    compute(0)
    wait_scatter(NCHUNK - 2, 1)
    pltpu.sync_copy(H[0], aggsh.at[didxr.at[(NCHUNK - 1) % 4]], add=True)

    plsc.subcore_barrier()
    _write_out(aggsh, agg_hbm, cid, sid)


def _edge_film(h, p, src, dst):
    return pl.kernel(
        _edge_film_body,
        out_type=jax.ShapeDtypeStruct((NC, N, D), jnp.float32),
        mesh=_MESH,
        compiler_params=_SC_PARAMS,
        scratch_types=[
            pltpu.VMEM((4, C), jnp.int32),
            pltpu.VMEM((4, C), jnp.int32),
            pltpu.VMEM((C, D), jnp.float32),
            pltpu.VMEM((C, D), jnp.float32),
            pltpu.VMEM((C, D), jnp.float32),
            pltpu.VMEM((C, D), jnp.float32),
            pltpu.VMEM_SHARED((N, D), jnp.float32),
            pltpu.SemaphoreType.DMA,
            pltpu.SemaphoreType.DMA,
            pltpu.SemaphoreType.DMA,
            pltpu.SemaphoreType.DMA,
            pltpu.SemaphoreType.DMA,
            pltpu.SemaphoreType.DMA,
            pltpu.SemaphoreType.DMA,
            pltpu.SemaphoreType.DMA,
        ],
    )(h, p, src, dst)


def _edge_sum_body(h_hbm, src_hbm, dst_hbm, agg_hbm,
                   sidxr, didxr, h0, h1, aggsh,
                   semh0, semh1, semi0, semi1, sems0, sems1):
    cid = lax.axis_index("c")
    sid = lax.axis_index("s")
    wid = sid * NC + cid
    H = (h0, h1)
    SEMH, SEMI, SEMS = (semh0, semh1), (semi0, semi1), (sems0, sems1)

    def base_of(k):
        return pl.multiple_of(wid * EPW + k * C, 8)

    def issue_idx(k, b):
        r = k % 4
        pltpu.async_copy(src_hbm.at[pl.ds(base_of(k), C)], sidxr.at[r],
                         SEMI[b])
        pltpu.async_copy(dst_hbm.at[pl.ds(base_of(k), C)], didxr.at[r],
                         SEMI[b])

    def wait_idx(k, b):
        r = k % 4
        pltpu.make_async_copy(src_hbm.at[pl.ds(base_of(k), C)], sidxr.at[r],
                              SEMI[b]).wait()
        pltpu.make_async_copy(dst_hbm.at[pl.ds(base_of(k), C)], didxr.at[r],
                              SEMI[b]).wait()

    def issue_gather(k, b):
        pltpu.async_copy(h_hbm.at[sidxr.at[k % 4]], H[b], SEMH[b])

    def wait_gather(k, b):
        pltpu.make_async_copy(h_hbm.at[sidxr.at[k % 4]], H[b],
                              SEMH[b]).wait()

    def issue_scatter(k, b):
        pltpu.async_copy(H[b], aggsh.at[didxr.at[k % 4]], SEMS[b], add=True)

    def wait_scatter(k, b):
        pltpu.make_async_copy(H[b], aggsh.at[didxr.at[k % 4]],
                              SEMS[b]).wait()

    def step(k, b, first):
        wait_gather(k, b)
        issue_scatter(k, b)

        @pl.when(k + 2 < NCHUNK)
        def _():
            issue_idx(k + 2, b)
        if not first:
            wait_scatter(k - 1, 1 - b)
        wait_idx(k + 1, 1 - b)
        issue_gather(k + 1, 1 - b)

    _zero_vmem_2d(h0, C)
    _zero_spmem(h0, aggsh, sid)
    plsc.subcore_barrier()

    issue_idx(0, 0)
    issue_idx(1, 1)
    wait_idx(0, 0)
    issue_gather(0, 0)
    step(0, 0, True)
    step(1, 1, False)

    def pair(kk, _):
        k = 2 * kk
        step(k, 0, False)
        step(k + 1, 1, False)
        return 0

    lax.fori_loop(1, NCHUNK // 2, pair, 0)
    wait_gather(NCHUNK - 1, 0)
    wait_scatter(NCHUNK - 2, 1)
    pltpu.sync_copy(H[0], aggsh.at[didxr.at[(NCHUNK - 1) % 4]], add=True)

    plsc.subcore_barrier()
    _write_out(aggsh, agg_hbm, cid, sid)


def _edge_sum(h, src, dst):
    return pl.kernel(
        _edge_sum_body,
        out_type=jax.ShapeDtypeStruct((NC, N, D), jnp.float32),
        mesh=_MESH,
        compiler_params=_SC_PARAMS,
        scratch_types=[
            pltpu.VMEM((4, C), jnp.int32),
            pltpu.VMEM((4, C), jnp.int32),
            pltpu.VMEM((C, D), jnp.float32),
            pltpu.VMEM((C, D), jnp.float32),
            pltpu.VMEM_SHARED((N, D), jnp.float32),
            pltpu.SemaphoreType.DMA,
            pltpu.SemaphoreType.DMA,
            pltpu.SemaphoreType.DMA,
            pltpu.SemaphoreType.DMA,
            pltpu.SemaphoreType.DMA,
            pltpu.SemaphoreType.DMA,
        ],
    )(h, src, dst)


def _cnt_body(dst_hbm, cnt_hbm, didx, cntv):
    cid = lax.axis_index("c")
    sid = lax.axis_index("s")
    wid = sid * NC + cid

    pltpu.sync_copy(dst_hbm.at[pl.ds(pl.multiple_of(wid * EPW, 8), EPW)], didx)

    def zc(i, _):
        cntv[pl.ds(i * L, L)] = jnp.zeros((L,), jnp.float32)
        return 0
    lax.fori_loop(0, N // L, zc, 0)

    ones16 = jnp.ones((L,), jnp.float32)

    def acc(g, _):
        plsc.addupdate_scatter(cntv, [didx[pl.ds(g * L, L)]], ones16)
        return 0
    lax.fori_loop(0, EPW // L, acc, 0)

    pltpu.sync_copy(cntv, cnt_hbm.at[wid, 0])


def _cnt(dst):
    return pl.kernel(
        _cnt_body,
        out_type=jax.ShapeDtypeStruct((NW, 1, N), jnp.float32),
        mesh=_MESH,
        compiler_params=_SC_PARAMS,
        scratch_types=[
            pltpu.VMEM((EPW,), jnp.int32),
            pltpu.VMEM((N,), jnp.float32),
        ],
    )(dst)


def kernel(x, edge_index, W0, Fw0, Fb0, Sw0, FSw0, W1, Fw1, Fb1, Sw1, FSw1,
           W2, Fw2, Fb2, Sw2, FSw2, bng0, bnb0, bng1, bnb1):
    src, dst = edge_index[0], edge_index[1]

    wcat0 = jnp.concatenate([Sw0, W0, Fw0, FSw0], axis=1)
    wcat1 = jnp.concatenate([Sw1, W1, Fw1, FSw1], axis=1)
    wcat2 = jnp.concatenate([Sw2, W2, Fw2, FSw2], axis=1)

    cnt_parts = _cnt(dst).reshape(NW, N)

    self0, h0, bg0, p0 = _dense(x, wcat0, Fb0.reshape(1, -1), act=True)
    agg0 = _edge_film(h0, p0, src, dst)

    self1, h1, bg1, p1 = _fused(self0, agg0, cnt_parts,
                                bng0.reshape(1, -1), bnb0.reshape(1, -1),
                                wcat1, Fb1.reshape(1, -1), act=True)
    agg1 = _edge_film(h1, p1, src, dst)

    self2, h2, bg2, p2 = _fused(self1, agg1, cnt_parts,
                                bng1.reshape(1, -1), bnb1.reshape(1, -1),
                                wcat2, Fb2.reshape(1, -1), act=False)
    agg2 = _edge_sum(h2, src, dst)
    return _combine2(self2, agg2, cnt_parts, bg2)


# speedup vs baseline: 1.2620x; 1.2446x over previous
import functools

import jax
import jax.numpy as jnp
from jax import lax
from jax.experimental import pallas as pl
from jax.experimental.pallas import tpu as pltpu
from jax.experimental.pallas import tpu_sc as plsc

N = 10000
E = 320000
D = 128
EPS = 1e-5

NC = 2
NS = 16
NW = NC * NS
L = 16

EPW = E // NW
C = 80
NCHUNK = EPW // C
ZCH = 80
NRCH = N // ZCH
KMAX = -(-NRCH // NS)

_SC_PARAMS = pltpu.CompilerParams(needs_layout_passes=False)


def _dense_body(act, x_ref, wcat_ref, fb_ref, self_ref, h_ref, bg_ref, p_ref):
    x = x_ref[...]
    y = jnp.dot(x, wcat_ref[...], preferred_element_type=jnp.float32)
    swy = y[:, :D]
    h = y[:, D:2 * D]
    bg = y[:, 2 * D:4 * D] + fb_ref[...]
    bgs = y[:, 4 * D:]
    beta_s, gamma_s = bgs[:, :D], bgs[:, D:]
    out = gamma_s * swy + beta_s
    if act:
        out = jnp.maximum(out, 0.0)
    self_ref[...] = out
    h_ref[...] = h
    bg_ref[...] = bg
    beta, gamma = bg[:, :D], bg[:, D:]
    b16 = lax.bitcast_convert_type(beta.astype(jnp.bfloat16),
                                   jnp.uint16).astype(jnp.uint32)
    g16 = lax.bitcast_convert_type(gamma.astype(jnp.bfloat16),
                                   jnp.uint16).astype(jnp.uint32)
    p_ref[...] = lax.bitcast_convert_type(b16 | (g16 << 16), jnp.float32)


def _dense(x, wcat, fb, act):
    blk = 1000
    grid = N // blk
    return pl.pallas_call(
        functools.partial(_dense_body, act),
        grid=(grid,),
        in_specs=[
            pl.BlockSpec((blk, D), lambda i: (i, 0)),
            pl.BlockSpec((D, 6 * D), lambda i: (0, 0)),
            pl.BlockSpec((1, 2 * D), lambda i: (0, 0)),
        ],
        out_specs=[
            pl.BlockSpec((blk, D), lambda i: (i, 0)),
            pl.BlockSpec((blk, D), lambda i: (i, 0)),
            pl.BlockSpec((blk, 2 * D), lambda i: (i, 0)),
            pl.BlockSpec((blk, D), lambda i: (i, 0)),
        ],
        out_shape=[
            jax.ShapeDtypeStruct((N, D), jnp.float32),
            jax.ShapeDtypeStruct((N, D), jnp.float32),
            jax.ShapeDtypeStruct((N, 2 * D), jnp.float32),
            jax.ShapeDtypeStruct((N, D), jnp.float32),
        ],
    )(x, wcat, fb)


def _fused_body(act, self_ref, agg_ref, cntp_ref, g_ref, b_ref,
                wcat_ref, fb_ref, self_o, h_o, bg_o, p_o):
    cnt = jnp.sum(cntp_ref[...], axis=0)
    recip = 1.0 / jnp.clip(cnt, 1.0, None)
    t = self_ref[...] + (agg_ref[0, :, :] + agg_ref[1, :, :]) * recip[:, None]
    m = jnp.mean(t, axis=0)
    v = jnp.mean((t - m) ** 2, axis=0)
    xn = g_ref[...] * (t - m) / jnp.sqrt(v + EPS) + b_ref[...]
    blk = 1000
    for i in range(N // blk):
        rows = pl.ds(i * blk, blk)
        y = jnp.dot(xn[i * blk:(i + 1) * blk, :], wcat_ref[...],
                    preferred_element_type=jnp.float32)
        swy = y[:, :D]
        h = y[:, D:2 * D]
        bg = y[:, 2 * D:4 * D] + fb_ref[...]
        bgs = y[:, 4 * D:]
        beta_s, gamma_s = bgs[:, :D], bgs[:, D:]
        out = gamma_s * swy + beta_s
        if act:
            out = jnp.maximum(out, 0.0)
        self_o[rows, :] = out
        h_o[rows, :] = h
        bg_o[rows, :] = bg
        beta, gamma = bg[:, :D], bg[:, D:]
        b16 = lax.bitcast_convert_type(beta.astype(jnp.bfloat16),
                                       jnp.uint16).astype(jnp.uint32)
        g16 = lax.bitcast_convert_type(gamma.astype(jnp.bfloat16),
                                       jnp.uint16).astype(jnp.uint32)
        p_o[rows, :] = lax.bitcast_convert_type(b16 | (g16 << 16),
                                                jnp.float32)


def _fused(selfo, agg, cnt_parts, g, b, wcat, fb, act):
    return pl.pallas_call(
        functools.partial(_fused_body, act),
        out_shape=[
            jax.ShapeDtypeStruct((N, D), jnp.float32),
            jax.ShapeDtypeStruct((N, D), jnp.float32),
            jax.ShapeDtypeStruct((N, 2 * D), jnp.float32),
            jax.ShapeDtypeStruct((N, D), jnp.float32),
        ],
    )(selfo, agg, cnt_parts, g, b, wcat, fb)


def _combine2_body(self_ref, agg_ref, cntp_ref, bg_ref, out_ref):
    cnt = jnp.sum(cntp_ref[...], axis=0)
    recip = 1.0 / jnp.clip(cnt, 1.0, None)
    ind = (cnt > 0.0).astype(jnp.float32)
    s = (agg_ref[0, :, :] + agg_ref[1, :, :]) * recip[:, None]
    beta, gamma = bg_ref[:, :D], bg_ref[:, D:]
    out_ref[...] = self_ref[...] + gamma * s + beta * ind[:, None]


def _combine2(selfo, agg, cnt_parts, bg):
    return pl.pallas_call(
        _combine2_body,
        out_shape=jax.ShapeDtypeStruct((N, D), jnp.float32),
    )(selfo, agg, cnt_parts, bg)


_MESH = plsc.VectorSubcoreMesh(core_axis_name="c", subcore_axis_name="s")


def _zero_vmem_2d(ref, rows):
    def body(i, _):
        for j in range(D // L):
            ref[i, pl.ds(j * L, L)] = jnp.zeros((L,), jnp.float32)
        return 0
    lax.fori_loop(0, rows, body, 0)


def _zero_spmem(zsrc, aggsh, sid):
    for k in range(KMAX):
        rc = k * NS + sid

        @pl.when(rc < NRCH)
        def _():
            pltpu.sync_copy(zsrc, aggsh.at[pl.ds(rc * ZCH, ZCH)])


def _write_out(aggsh, agg_hbm, cid, sid):
    for k in range(KMAX):
        rc = k * NS + sid

        @pl.when(rc < NRCH)
        def _():
            r0 = rc * ZCH
            pltpu.sync_copy(aggsh.at[pl.ds(r0, ZCH)],
                            agg_hbm.at[cid, pl.ds(r0, ZCH)])


def _edge_film_body(h_hbm, p_hbm, src_hbm, dst_hbm, agg_hbm,
                    sidxr, didxr, h0, h1, p0, p1, aggsh,
                    semh0, semh1, semp0, semp1, semi0, semi1, sems0, sems1):
    cid = lax.axis_index("c")
    sid = lax.axis_index("s")
    wid = sid * NC + cid
    H, P = (h0, h1), (p0, p1)
    SEMH, SEMP = (semh0, semh1), (semp0, semp1)
    SEMI, SEMS = (semi0, semi1), (sems0, sems1)

    def base_of(k):
        return pl.multiple_of(wid * EPW + k * C, 8)

    def issue_idx(k, b):
        r = k % 4
        pltpu.async_copy(src_hbm.at[pl.ds(base_of(k), C)], sidxr.at[r],
                         SEMI[b])
        pltpu.async_copy(dst_hbm.at[pl.ds(base_of(k), C)], didxr.at[r],
                         SEMI[b])

    def wait_idx(k, b):
        r = k % 4
        pltpu.make_async_copy(src_hbm.at[pl.ds(base_of(k), C)], sidxr.at[r],
                              SEMI[b]).wait()
        pltpu.make_async_copy(dst_hbm.at[pl.ds(base_of(k), C)], didxr.at[r],
                              SEMI[b]).wait()

    def issue_gather(k, b):
        r = k % 4
        pltpu.async_copy(h_hbm.at[sidxr.at[r]], H[b], SEMH[b])
        pltpu.async_copy(p_hbm.at[didxr.at[r]], P[b], SEMP[b])

    def wait_gather(k, b):
        r = k % 4
        pltpu.make_async_copy(h_hbm.at[sidxr.at[r]], H[b], SEMH[b]).wait()
        pltpu.make_async_copy(p_hbm.at[didxr.at[r]], P[b], SEMP[b]).wait()

    def issue_scatter(k, b):
        pltpu.async_copy(H[b], aggsh.at[didxr.at[k % 4]], SEMS[b], add=True)

    def wait_scatter(k, b):
        pltpu.make_async_copy(H[b], aggsh.at[didxr.at[k % 4]],
                              SEMS[b]).wait()

    def compute(b):
        @plsc.parallel_loop(0, C, step=1, unroll=2)
        def _(i):
            for j in range(D // L):
                pw = P[b][i, pl.ds(L * j, L)]
                pj = plsc.bitcast(pw, jnp.bfloat16)
                bb, gg = plsc.unpack(pj, format=plsc.PackFormat.INTERLEAVED,
                                     preferred_element_type=jnp.float32)
                hh = H[b][i, pl.ds(L * j, L)]
                H[b][i, pl.ds(L * j, L)] = jnp.maximum(gg * hh + bb, 0.0)

    def step(k, b, first):
        wait_gather(k, b)
        if not first:
            wait_scatter(k - 1, 1 - b)
        wait_idx(k + 1, 1 - b)
        issue_gather(k + 1, 1 - b)

        @pl.when(k + 2 < NCHUNK)
        def _():
            issue_idx(k + 2, b)
        compute(b)
        issue_scatter(k, b)

    _zero_vmem_2d(h0, C)
    _zero_spmem(h0, aggsh, sid)
    plsc.subcore_barrier()

    issue_idx(0, 0)
    issue_idx(1, 1)
    wait_idx(0, 0)
    issue_gather(0, 0)
    step(0, 0, True)
    step(1, 1, False)

    def pair(kk, _):
        k = 2 * kk
        step(k, 0, False)
        step(k + 1, 1, False)
        return 0

    lax.fori_loop(1, NCHUNK // 2, pair, 0)
    wait_gather(NCHUNK - 1, 0)
    compute(0)
    wait_scatter(NCHUNK - 2, 1)
    pltpu.sync_copy(H[0], aggsh.at[didxr.at[(NCHUNK - 1) % 4]], add=True)

    plsc.subcore_barrier()
    _write_out(aggsh, agg_hbm, cid, sid)


def _edge_film(h, p, src, dst):
    return pl.kernel(
        _edge_film_body,
        out_type=jax.ShapeDtypeStruct((NC, N, D), jnp.float32),
        mesh=_MESH,
        compiler_params=_SC_PARAMS,
        scratch_types=[
            pltpu.VMEM((4, C), jnp.int32),
            pltpu.VMEM((4, C), jnp.int32),
            pltpu.VMEM((C, D), jnp.float32),
            pltpu.VMEM((C, D), jnp.float32),
            pltpu.VMEM((C, D), jnp.float32),
            pltpu.VMEM((C, D), jnp.float32),
            pltpu.VMEM_SHARED((N, D), jnp.float32),
            pltpu.SemaphoreType.DMA,
            pltpu.SemaphoreType.DMA,
            pltpu.SemaphoreType.DMA,
            pltpu.SemaphoreType.DMA,
            pltpu.SemaphoreType.DMA,
            pltpu.SemaphoreType.DMA,
            pltpu.SemaphoreType.DMA,
            pltpu.SemaphoreType.DMA,
        ],
    )(h, p, src, dst)


def _edge_sum_body(h_hbm, src_hbm, dst_hbm, agg_hbm,
                   sidxr, didxr, h0, h1, aggsh,
                   semh0, semh1, semi0, semi1, sems0, sems1):
    cid = lax.axis_index("c")
    sid = lax.axis_index("s")
    wid = sid * NC + cid
    H = (h0, h1)
    SEMH, SEMI, SEMS = (semh0, semh1), (semi0, semi1), (sems0, sems1)

    def base_of(k):
        return pl.multiple_of(wid * EPW + k * C, 8)

    def issue_idx(k, b):
        r = k % 4
        pltpu.async_copy(src_hbm.at[pl.ds(base_of(k), C)], sidxr.at[r],
                         SEMI[b])
        pltpu.async_copy(dst_hbm.at[pl.ds(base_of(k), C)], didxr.at[r],
                         SEMI[b])

    def wait_idx(k, b):
        r = k % 4
        pltpu.make_async_copy(src_hbm.at[pl.ds(base_of(k), C)], sidxr.at[r],
                              SEMI[b]).wait()
        pltpu.make_async_copy(dst_hbm.at[pl.ds(base_of(k), C)], didxr.at[r],
                              SEMI[b]).wait()

    def issue_gather(k, b):
        pltpu.async_copy(h_hbm.at[sidxr.at[k % 4]], H[b], SEMH[b])

    def wait_gather(k, b):
        pltpu.make_async_copy(h_hbm.at[sidxr.at[k % 4]], H[b],
                              SEMH[b]).wait()

    def issue_scatter(k, b):
        pltpu.async_copy(H[b], aggsh.at[didxr.at[k % 4]], SEMS[b], add=True)

    def wait_scatter(k, b):
        pltpu.make_async_copy(H[b], aggsh.at[didxr.at[k % 4]],
                              SEMS[b]).wait()

    def step(k, b, first):
        wait_gather(k, b)
        if not first:
            wait_scatter(k - 1, 1 - b)
        wait_idx(k + 1, 1 - b)
        issue_gather(k + 1, 1 - b)

        @pl.when(k + 2 < NCHUNK)
        def _():
            issue_idx(k + 2, b)
        issue_scatter(k, b)

    _zero_vmem_2d(h0, C)
    _zero_spmem(h0, aggsh, sid)
    plsc.subcore_barrier()

    issue_idx(0, 0)
    issue_idx(1, 1)
    wait_idx(0, 0)
    issue_gather(0, 0)
    step(0, 0, True)
    step(1, 1, False)

    def pair(kk, _):
        k = 2 * kk
        step(k, 0, False)
        step(k + 1, 1, False)
        return 0

    lax.fori_loop(1, NCHUNK // 2, pair, 0)
    wait_gather(NCHUNK - 1, 0)
    wait_scatter(NCHUNK - 2, 1)
    pltpu.sync_copy(H[0], aggsh.at[didxr.at[(NCHUNK - 1) % 4]], add=True)

    plsc.subcore_barrier()
    _write_out(aggsh, agg_hbm, cid, sid)


def _edge_sum(h, src, dst):
    return pl.kernel(
        _edge_sum_body,
        out_type=jax.ShapeDtypeStruct((NC, N, D), jnp.float32),
        mesh=_MESH,
        compiler_params=_SC_PARAMS,
        scratch_types=[
            pltpu.VMEM((4, C), jnp.int32),
            pltpu.VMEM((4, C), jnp.int32),
            pltpu.VMEM((C, D), jnp.float32),
            pltpu.VMEM((C, D), jnp.float32),
            pltpu.VMEM_SHARED((N, D), jnp.float32),
            pltpu.SemaphoreType.DMA,
            pltpu.SemaphoreType.DMA,
            pltpu.SemaphoreType.DMA,
            pltpu.SemaphoreType.DMA,
            pltpu.SemaphoreType.DMA,
            pltpu.SemaphoreType.DMA,
        ],
    )(h, src, dst)


def _cnt_body(dst_hbm, cnt_hbm, didx, cntv):
    cid = lax.axis_index("c")
    sid = lax.axis_index("s")
    wid = sid * NC + cid

    pltpu.sync_copy(dst_hbm.at[pl.ds(pl.multiple_of(wid * EPW, 8), EPW)], didx)

    def zc(i, _):
        cntv[pl.ds(i * L, L)] = jnp.zeros((L,), jnp.float32)
        return 0
    lax.fori_loop(0, N // L, zc, 0)

    ones16 = jnp.ones((L,), jnp.float32)

    def acc(g, _):
        plsc.addupdate_scatter(cntv, [didx[pl.ds(g * L, L)]], ones16)
        return 0
    lax.fori_loop(0, EPW // L, acc, 0)

    pltpu.sync_copy(cntv, cnt_hbm.at[wid, 0])


def _cnt(dst):
    return pl.kernel(
        _cnt_body,
        out_type=jax.ShapeDtypeStruct((NW, 1, N), jnp.float32),
        mesh=_MESH,
        compiler_params=_SC_PARAMS,
        scratch_types=[
            pltpu.VMEM((EPW,), jnp.int32),
            pltpu.VMEM((N,), jnp.float32),
        ],
    )(dst)


def kernel(x, edge_index, W0, Fw0, Fb0, Sw0, FSw0, W1, Fw1, Fb1, Sw1, FSw1,
           W2, Fw2, Fb2, Sw2, FSw2, bng0, bnb0, bng1, bnb1):
    src, dst = edge_index[0], edge_index[1]

    wcat0 = jnp.concatenate([Sw0, W0, Fw0, FSw0], axis=1)
    wcat1 = jnp.concatenate([Sw1, W1, Fw1, FSw1], axis=1)
    wcat2 = jnp.concatenate([Sw2, W2, Fw2, FSw2], axis=1)

    cnt_parts = _cnt(dst).reshape(NW, N)

    self0, h0, bg0, p0 = _dense(x, wcat0, Fb0.reshape(1, -1), act=True)
    agg0 = _edge_film(h0, p0, src, dst)

    self1, h1, bg1, p1 = _fused(self0, agg0, cnt_parts,
                                bng0.reshape(1, -1), bnb0.reshape(1, -1),
                                wcat1, Fb1.reshape(1, -1), act=True)
    agg1 = _edge_film(h1, p1, src, dst)

    self2, h2, bg2, p2 = _fused(self1, agg1, cnt_parts,
                                bng1.reshape(1, -1), bnb1.reshape(1, -1),
                                wcat2, Fb2.reshape(1, -1), act=False)
    agg2 = _edge_sum(h2, src, dst)
    return _combine2(self2, agg2, cnt_parts, bg2)


# speedup vs baseline: 1.2899x; 1.0221x over previous
import functools

import jax
import jax.numpy as jnp
from jax import lax
from jax.experimental import pallas as pl
from jax.experimental.pallas import tpu as pltpu
from jax.experimental.pallas import tpu_sc as plsc

N = 10000
E = 320000
D = 128
EPS = 1e-5

NC = 2
NS = 16
NW = NC * NS
L = 16

EPW = E // NW
C = 80
NCHUNK = EPW // C
ZCH = 80
NRCH = N // ZCH
KMAX = -(-NRCH // NS)

_SC_PARAMS = pltpu.CompilerParams(needs_layout_passes=False)


def _pack_bg(bg):
    beta, gamma = bg[:, :D], bg[:, D:]
    b16 = lax.bitcast_convert_type(beta.astype(jnp.bfloat16),
                                   jnp.uint16).astype(jnp.uint32)
    g16 = lax.bitcast_convert_type(gamma.astype(jnp.bfloat16),
                                   jnp.uint16).astype(jnp.uint32)
    return lax.bitcast_convert_type(b16 | (g16 << 16), jnp.float32)


def _dense_body(act, x_ref, w_ref, fw_ref, sw_ref, fsw_ref, fb_ref,
                self_ref, h_ref, p_ref):
    x = x_ref[...]
    swy = jnp.dot(x, sw_ref[...], preferred_element_type=jnp.float32)
    h = jnp.dot(x, w_ref[...], preferred_element_type=jnp.float32)
    bg = jnp.dot(x, fw_ref[...], preferred_element_type=jnp.float32) + fb_ref[...]
    bgs = jnp.dot(x, fsw_ref[...], preferred_element_type=jnp.float32)
    beta_s, gamma_s = bgs[:, :D], bgs[:, D:]
    out = gamma_s * swy + beta_s
    if act:
        out = jnp.maximum(out, 0.0)
    self_ref[...] = out
    h_ref[...] = h
    p_ref[...] = _pack_bg(bg)


def _dense(x, w, fw, sw, fsw, fb, act):
    blk = 1000
    grid = N // blk
    return pl.pallas_call(
        functools.partial(_dense_body, act),
        grid=(grid,),
        in_specs=[
            pl.BlockSpec((blk, D), lambda i: (i, 0)),
            pl.BlockSpec((D, D), lambda i: (0, 0)),
            pl.BlockSpec((D, 2 * D), lambda i: (0, 0)),
            pl.BlockSpec((D, D), lambda i: (0, 0)),
            pl.BlockSpec((D, 2 * D), lambda i: (0, 0)),
            pl.BlockSpec((1, 2 * D), lambda i: (0, 0)),
        ],
        out_specs=[
            pl.BlockSpec((blk, D), lambda i: (i, 0)),
            pl.BlockSpec((blk, D), lambda i: (i, 0)),
            pl.BlockSpec((blk, D), lambda i: (i, 0)),
        ],
        out_shape=[
            jax.ShapeDtypeStruct((N, D), jnp.float32),
            jax.ShapeDtypeStruct((N, D), jnp.float32),
            jax.ShapeDtypeStruct((N, D), jnp.float32),
        ],
    )(x, w, fw, sw, fsw, fb)


def _fused_body(act, emit_bg, self_ref, agg_ref, cntp_ref, g_ref, b_ref,
                w_ref, fw_ref, sw_ref, fsw_ref, fb_ref,
                self_o, h_o, p_o, *rest):
    cnt = jnp.sum(cntp_ref[...], axis=0)
    recip = 1.0 / jnp.clip(cnt, 1.0, None)
    t = self_ref[...] + (agg_ref[0, :, :] + agg_ref[1, :, :]) * recip[:, None]
    m = jnp.mean(t, axis=0)
    v = jnp.mean((t - m) ** 2, axis=0)
    xn = g_ref[...] * (t - m) / jnp.sqrt(v + EPS) + b_ref[...]
    blk = 1000
    for i in range(N // blk):
        rows = pl.ds(i * blk, blk)
        xb = xn[i * blk:(i + 1) * blk, :]
        swy = jnp.dot(xb, sw_ref[...], preferred_element_type=jnp.float32)
        h = jnp.dot(xb, w_ref[...], preferred_element_type=jnp.float32)
        bg = jnp.dot(xb, fw_ref[...],
                     preferred_element_type=jnp.float32) + fb_ref[...]
        bgs = jnp.dot(xb, fsw_ref[...], preferred_element_type=jnp.float32)
        beta_s, gamma_s = bgs[:, :D], bgs[:, D:]
        out = gamma_s * swy + beta_s
        if act:
            out = jnp.maximum(out, 0.0)
        self_o[rows, :] = out
        h_o[rows, :] = h
        p_o[rows, :] = _pack_bg(bg)
        if emit_bg:
            rest[0][rows, :] = bg


def _fused(selfo, agg, cnt_parts, g, b, w, fw, sw, fsw, fb, act, emit_bg):
    out_shape = [
        jax.ShapeDtypeStruct((N, D), jnp.float32),
        jax.ShapeDtypeStruct((N, D), jnp.float32),
        jax.ShapeDtypeStruct((N, D), jnp.float32),
    ]
    if emit_bg:
        out_shape.append(jax.ShapeDtypeStruct((N, 2 * D), jnp.float32))
    return pl.pallas_call(
        functools.partial(_fused_body, act, emit_bg),
        out_shape=out_shape,
    )(selfo, agg, cnt_parts, g, b, w, fw, sw, fsw, fb)


def _combine2_body(self_ref, agg_ref, cntp_ref, bg_ref, out_ref):
    cnt = jnp.sum(cntp_ref[...], axis=0)
    recip = 1.0 / jnp.clip(cnt, 1.0, None)
    ind = (cnt > 0.0).astype(jnp.float32)
    s = (agg_ref[0, :, :] + agg_ref[1, :, :]) * recip[:, None]
    beta, gamma = bg_ref[:, :D], bg_ref[:, D:]
    out_ref[...] = self_ref[...] + gamma * s + beta * ind[:, None]


def _combine2(selfo, agg, cnt_parts, bg):
    return pl.pallas_call(
        _combine2_body,
        out_shape=jax.ShapeDtypeStruct((N, D), jnp.float32),
    )(selfo, agg, cnt_parts, bg)


_MESH = plsc.VectorSubcoreMesh(core_axis_name="c", subcore_axis_name="s")


def _zero_vmem_2d(ref, rows):
    def body(i, _):
        for j in range(D // L):
            ref[i, pl.ds(j * L, L)] = jnp.zeros((L,), jnp.float32)
        return 0
    lax.fori_loop(0, rows, body, 0)


def _zero_spmem(zsrc, aggsh, sid):
    for k in range(KMAX):
        rc = k * NS + sid

        @pl.when(rc < NRCH)
        def _():
            pltpu.sync_copy(zsrc, aggsh.at[pl.ds(rc * ZCH, ZCH)])


def _write_out(aggsh, agg_hbm, cid, sid):
    for k in range(KMAX):
        rc = k * NS + sid

        @pl.when(rc < NRCH)
        def _():
            r0 = rc * ZCH
            pltpu.sync_copy(aggsh.at[pl.ds(r0, ZCH)],
                            agg_hbm.at[cid, pl.ds(r0, ZCH)])


def _edge_film_body(h_hbm, p_hbm, ei_hbm, agg_hbm,
                    sidxr, didxr, h0, h1, p0, p1, aggsh,
                    semh0, semh1, semp0, semp1, semi0, semi1, sems0, sems1):
    cid = lax.axis_index("c")
    sid = lax.axis_index("s")
    wid = sid * NC + cid
    H, P = (h0, h1), (p0, p1)
    SEMH, SEMP = (semh0, semh1), (semp0, semp1)
    SEMI, SEMS = (semi0, semi1), (sems0, sems1)

    def base_of(k):
        return pl.multiple_of(wid * EPW + k * C, 8)

    def issue_idx(k, b):
        r = k % 4
        pltpu.async_copy(ei_hbm.at[pl.ds(base_of(k), C)], sidxr.at[r],
                         SEMI[b])
        pltpu.async_copy(ei_hbm.at[pl.ds(E + base_of(k), C)], didxr.at[r],
                         SEMI[b])

    def wait_idx(k, b):
        r = k % 4
        pltpu.make_async_copy(ei_hbm.at[pl.ds(base_of(k), C)],
                              sidxr.at[r], SEMI[b]).wait()
        pltpu.make_async_copy(ei_hbm.at[pl.ds(E + base_of(k), C)],
                              didxr.at[r], SEMI[b]).wait()

    def issue_gather(k, b):
        r = k % 4
        pltpu.async_copy(h_hbm.at[sidxr.at[r]], H[b], SEMH[b])
        pltpu.async_copy(p_hbm.at[didxr.at[r]], P[b], SEMP[b])

    def wait_gather(k, b):
        r = k % 4
        pltpu.make_async_copy(h_hbm.at[sidxr.at[r]], H[b], SEMH[b]).wait()
        pltpu.make_async_copy(p_hbm.at[didxr.at[r]], P[b], SEMP[b]).wait()

    def issue_scatter(k, b):
        pltpu.async_copy(H[b], aggsh.at[didxr.at[k % 4]], SEMS[b], add=True)

    def wait_scatter(k, b):
        pltpu.make_async_copy(H[b], aggsh.at[didxr.at[k % 4]],
                              SEMS[b]).wait()

    def compute(b):
        @plsc.parallel_loop(0, C, step=1, unroll=2)
        def _(i):
            for j in range(D // L):
                pw = P[b][i, pl.ds(L * j, L)]
                pj = plsc.bitcast(pw, jnp.bfloat16)
                bb, gg = plsc.unpack(pj, format=plsc.PackFormat.INTERLEAVED,
                                     preferred_element_type=jnp.float32)
                hh = H[b][i, pl.ds(L * j, L)]
                H[b][i, pl.ds(L * j, L)] = jnp.maximum(gg * hh + bb, 0.0)

    def step(k, b, first):
        wait_gather(k, b)
        if not first:
            wait_scatter(k - 1, 1 - b)
        wait_idx(k + 1, 1 - b)
        issue_gather(k + 1, 1 - b)

        @pl.when(k + 2 < NCHUNK)
        def _():
            issue_idx(k + 2, b)
        compute(b)
        issue_scatter(k, b)

    _zero_vmem_2d(h0, C)
    _zero_spmem(h0, aggsh, sid)
    plsc.subcore_barrier()

    issue_idx(0, 0)
    issue_idx(1, 1)
    wait_idx(0, 0)
    issue_gather(0, 0)
    step(0, 0, True)
    step(1, 1, False)

    def pair(kk, _):
        k = 2 * kk
        step(k, 0, False)
        step(k + 1, 1, False)
        return 0

    lax.fori_loop(1, NCHUNK // 2, pair, 0)
    wait_gather(NCHUNK - 1, 0)
    compute(0)
    wait_scatter(NCHUNK - 2, 1)
    pltpu.sync_copy(H[0], aggsh.at[didxr.at[(NCHUNK - 1) % 4]], add=True)

    plsc.subcore_barrier()
    _write_out(aggsh, agg_hbm, cid, sid)


def _edge_film(h, p, ei):
    return pl.kernel(
        _edge_film_body,
        out_type=jax.ShapeDtypeStruct((NC, N, D), jnp.float32),
        mesh=_MESH,
        compiler_params=_SC_PARAMS,
        scratch_types=[
            pltpu.VMEM((4, C), jnp.int32),
            pltpu.VMEM((4, C), jnp.int32),
            pltpu.VMEM((C, D), jnp.float32),
            pltpu.VMEM((C, D), jnp.float32),
            pltpu.VMEM((C, D), jnp.float32),
            pltpu.VMEM((C, D), jnp.float32),
            pltpu.VMEM_SHARED((N, D), jnp.float32),
            pltpu.SemaphoreType.DMA,
            pltpu.SemaphoreType.DMA,
            pltpu.SemaphoreType.DMA,
            pltpu.SemaphoreType.DMA,
            pltpu.SemaphoreType.DMA,
            pltpu.SemaphoreType.DMA,
            pltpu.SemaphoreType.DMA,
            pltpu.SemaphoreType.DMA,
        ],
    )(h, p, ei)


def _edge_sum_body(h_hbm, ei_hbm, agg_hbm,
                   sidxr, didxr, h0, h1, aggsh,
                   semh0, semh1, semi0, semi1, sems0, sems1):
    cid = lax.axis_index("c")
    sid = lax.axis_index("s")
    wid = sid * NC + cid
    H = (h0, h1)
    SEMH, SEMI, SEMS = (semh0, semh1), (semi0, semi1), (sems0, sems1)

    def base_of(k):
        return pl.multiple_of(wid * EPW + k * C, 8)

    def issue_idx(k, b):
        r = k % 4
        pltpu.async_copy(ei_hbm.at[pl.ds(base_of(k), C)], sidxr.at[r],
                         SEMI[b])
        pltpu.async_copy(ei_hbm.at[pl.ds(E + base_of(k), C)], didxr.at[r],
                         SEMI[b])

    def wait_idx(k, b):
        r = k % 4
        pltpu.make_async_copy(ei_hbm.at[pl.ds(base_of(k), C)],
                              sidxr.at[r], SEMI[b]).wait()
        pltpu.make_async_copy(ei_hbm.at[pl.ds(E + base_of(k), C)],
                              didxr.at[r], SEMI[b]).wait()

    def issue_gather(k, b):
        pltpu.async_copy(h_hbm.at[sidxr.at[k % 4]], H[b], SEMH[b])

    def wait_gather(k, b):
        pltpu.make_async_copy(h_hbm.at[sidxr.at[k % 4]], H[b],
                              SEMH[b]).wait()

    def issue_scatter(k, b):
        pltpu.async_copy(H[b], aggsh.at[didxr.at[k % 4]], SEMS[b], add=True)

    def wait_scatter(k, b):
        pltpu.make_async_copy(H[b], aggsh.at[didxr.at[k % 4]],
                              SEMS[b]).wait()

    def step(k, b, first):
        wait_gather(k, b)
        if not first:
            wait_scatter(k - 1, 1 - b)
        wait_idx(k + 1, 1 - b)
        issue_gather(k + 1, 1 - b)

        @pl.when(k + 2 < NCHUNK)
        def _():
            issue_idx(k + 2, b)
        issue_scatter(k, b)

    _zero_vmem_2d(h0, C)
    _zero_spmem(h0, aggsh, sid)
    plsc.subcore_barrier()

    issue_idx(0, 0)
    issue_idx(1, 1)
    wait_idx(0, 0)
    issue_gather(0, 0)
    step(0, 0, True)
    step(1, 1, False)

    def pair(kk, _):
        k = 2 * kk
        step(k, 0, False)
        step(k + 1, 1, False)
        return 0

    lax.fori_loop(1, NCHUNK // 2, pair, 0)
    wait_gather(NCHUNK - 1, 0)
    wait_scatter(NCHUNK - 2, 1)
    pltpu.sync_copy(H[0], aggsh.at[didxr.at[(NCHUNK - 1) % 4]], add=True)

    plsc.subcore_barrier()
    _write_out(aggsh, agg_hbm, cid, sid)


def _edge_sum(h, ei):
    return pl.kernel(
        _edge_sum_body,
        out_type=jax.ShapeDtypeStruct((NC, N, D), jnp.float32),
        mesh=_MESH,
        compiler_params=_SC_PARAMS,
        scratch_types=[
            pltpu.VMEM((4, C), jnp.int32),
            pltpu.VMEM((4, C), jnp.int32),
            pltpu.VMEM((C, D), jnp.float32),
            pltpu.VMEM((C, D), jnp.float32),
            pltpu.VMEM_SHARED((N, D), jnp.float32),
            pltpu.SemaphoreType.DMA,
            pltpu.SemaphoreType.DMA,
            pltpu.SemaphoreType.DMA,
            pltpu.SemaphoreType.DMA,
            pltpu.SemaphoreType.DMA,
            pltpu.SemaphoreType.DMA,
        ],
    )(h, ei)


def _cnt_body(ei_hbm, cnt_hbm, didx, cntv):
    cid = lax.axis_index("c")
    sid = lax.axis_index("s")
    wid = sid * NC + cid

    pltpu.sync_copy(
        ei_hbm.at[pl.ds(pl.multiple_of(E + wid * EPW, 8), EPW)], didx)

    def zc(i, _):
        cntv[pl.ds(i * L, L)] = jnp.zeros((L,), jnp.float32)
        return 0
    lax.fori_loop(0, N // L, zc, 0)

    ones16 = jnp.ones((L,), jnp.float32)

    def acc(g, _):
        plsc.addupdate_scatter(cntv, [didx[pl.ds(g * L, L)]], ones16)
        return 0
    lax.fori_loop(0, EPW // L, acc, 0)

    pltpu.sync_copy(cntv, cnt_hbm.at[wid, 0])


def _cnt(ei):
    return pl.kernel(
        _cnt_body,
        out_type=jax.ShapeDtypeStruct((NW, 1, N), jnp.float32),
        mesh=_MESH,
        compiler_params=_SC_PARAMS,
        scratch_types=[
            pltpu.VMEM((EPW,), jnp.int32),
            pltpu.VMEM((N,), jnp.float32),
        ],
    )(ei)


def kernel(x, edge_index, W0, Fw0, Fb0, Sw0, FSw0, W1, Fw1, Fb1, Sw1, FSw1,
           W2, Fw2, Fb2, Sw2, FSw2, bng0, bnb0, bng1, bnb1):
    ei = edge_index.reshape(2 * E)

    cnt_parts = _cnt(ei).reshape(NW, N)

    self0, h0, p0 = _dense(x, W0, Fw0, Sw0, FSw0, Fb0.reshape(1, -1),
                           act=True)
    agg0 = _edge_film(h0, p0, ei)

    self1, h1, p1 = _fused(self0, agg0, cnt_parts,
                           bng0.reshape(1, -1), bnb0.reshape(1, -1),
                           W1, Fw1, Sw1, FSw1, Fb1.reshape(1, -1),
                           act=True, emit_bg=False)
    agg1 = _edge_film(h1, p1, ei)

    self2, h2, p2, bg2 = _fused(self1, agg1, cnt_parts,
                                bng1.reshape(1, -1), bnb1.reshape(1, -1),
                                W2, Fw2, Sw2, FSw2, Fb2.reshape(1, -1),
                                act=False, emit_bg=True)
    agg2 = _edge_sum(h2, ei)
    return _combine2(self2, agg2, cnt_parts, bg2)
